# Initial kernel scaffold; baseline (speedup 1.0000x reference)
import jax
import jax.numpy as jnp
from jax.experimental import pallas as pl


def kernel(x, edge_index, W1, b1, W2, b2, W3, b3, W4, b4):
    raise NotImplementedError("write your pallas kernel here")


# speedup vs baseline: 12.8696x; 12.8696x over previous
import functools

import jax
import jax.numpy as jnp
from jax import lax
from jax.experimental import pallas as pl
from jax.experimental.pallas import tpu as pltpu
from jax.experimental.pallas import tpu_sc as plsc

N = 10000
NP = 10112
E = 320000
NW = 32
CH = 128
TPC = 80
EP = NW * TPC * CH
RS = NP // 16

_mesh = plsc.VectorSubcoreMesh(core_axis_name="c", subcore_axis_name="s")
_sc_params = pltpu.CompilerParams(use_tc_tiling_on_sc=False)


@functools.partial(
    pl.kernel,
    out_type=jax.ShapeDtypeStruct((2, NP, 16), jnp.float32),
    mesh=_mesh,
    compiler_params=_sc_params,
    scratch_types=[
        pltpu.VMEM((TPC, CH), jnp.int32),
        pltpu.VMEM((CH, 16), jnp.float32),
        pltpu.VMEM_SHARED((NP, 16), jnp.float32),
    ],
)
def _deg_sc(dst_hbm, zeros_hbm, ones_hbm, out_hbm, dstv, onesv, deg_sp):
    c = lax.axis_index("c")
    s = lax.axis_index("s")
    w = c * 16 + s
    pltpu.sync_copy(zeros_hbm.at[pl.ds(s * RS, RS)], deg_sp.at[pl.ds(s * RS, RS)])
    pltpu.sync_copy(ones_hbm, onesv)
    pltpu.sync_copy(dst_hbm.at[w], dstv)
    plsc.subcore_barrier()

    def body(j, carry):
        pltpu.sync_copy(onesv, deg_sp.at[dstv.at[j]], add=True)
        return carry

    lax.fori_loop(0, TPC, body, 0)
    plsc.subcore_barrier()
    pltpu.sync_copy(deg_sp.at[pl.ds(s * RS, RS)], out_hbm.at[c, pl.ds(s * RS, RS)])


@functools.partial(
    pl.kernel,
    out_type=jax.ShapeDtypeStruct((2, NP, 64), jnp.float32),
    mesh=_mesh,
    compiler_params=_sc_params,
    scratch_types=[
        pltpu.VMEM((TPC, CH), jnp.int32),
        pltpu.VMEM((TPC, CH), jnp.int32),
        pltpu.VMEM((2, CH, 64), jnp.float32),
        pltpu.VMEM_SHARED((NP, 64), jnp.float32),
        pltpu.SemaphoreType.DMA,
        pltpu.SemaphoreType.DMA,
    ],
)
def _agg_sc(hs_hbm, src_hbm, dst_hbm, zeros_hbm, out_hbm,
            srcv, dstv, rows, agg_sp, sem0, sem1):
    c = lax.axis_index("c")
    s = lax.axis_index("s")
    w = c * 16 + s
    pltpu.sync_copy(zeros_hbm.at[pl.ds(s * RS, RS)], agg_sp.at[pl.ds(s * RS, RS)])
    pltpu.sync_copy(src_hbm.at[w], srcv)
    pltpu.sync_copy(dst_hbm.at[w], dstv)
    plsc.subcore_barrier()

    sems = (sem0, sem1)

    def gather(j, b):
        return pltpu.make_async_copy(hs_hbm.at[srcv.at[j]], rows.at[b], sems[b])

    gather(0, 0).start()
    gather(1, 1).start()

    def body(i, carry):
        for b in range(2):
            j = i * 2 + b
            gather(j, b).wait()
            pltpu.sync_copy(rows.at[b], agg_sp.at[dstv.at[j]], add=True)

            @pl.when(j + 2 < TPC)
            def _():
                gather(j + 2, b).start()
        return carry

    lax.fori_loop(0, TPC // 2, body, 0)
    plsc.subcore_barrier()
    pltpu.sync_copy(agg_sp.at[pl.ds(s * RS, RS)], out_hbm.at[c, pl.ds(s * RS, RS)])


def _tc_first(x_ref, w_ref, degp_ref, hs_ref, dinv_ref):
    deg = degp_ref[0, :, 0:1] + degp_ref[1, :, 0:1] + 1.0
    rows = lax.broadcasted_iota(jnp.int32, (NP, 1), 0)
    dinv = jnp.where(rows < N, lax.rsqrt(deg), 0.0)
    t = jnp.dot(x_ref[...], w_ref[...], preferred_element_type=jnp.float32)
    hs_ref[...] = t * dinv
    dinv_ref[...] = jnp.broadcast_to(dinv, (NP, 16))


def _tc_mid(p_ref, hs_ref, dinv_ref, b_ref, w_ref, out_ref):
    dinv = dinv_ref[:, 0:1]
    agg = p_ref[0] + p_ref[1] + hs_ref[...]
    h = jnp.maximum(agg * dinv + b_ref[...], 0.0)
    out_ref[...] = jnp.dot(h, w_ref[...], preferred_element_type=jnp.float32) * dinv


def _tc_pre4(p_ref, hs_ref, dinv_ref, b_ref, out_ref):
    dinv = dinv_ref[:, 0:1]
    agg = p_ref[0] + p_ref[1] + hs_ref[...]
    out_ref[...] = jnp.maximum(agg * dinv + b_ref[...], 0.0) * dinv


def _tc_last(p_ref, hs_ref, dinv_ref, w_ref, b_ref, out_ref):
    dinv = dinv_ref[:, 0:1]
    agg = (p_ref[0] + p_ref[1] + hs_ref[...]) * dinv
    out_ref[...] = jnp.dot(agg, w_ref[...], preferred_element_type=jnp.float32) + b_ref[...]


def _f32(*shape):
    return jax.ShapeDtypeStruct(shape, jnp.float32)


def kernel(x, edge_index, W1, b1, W2, b2, W3, b3, W4, b4):
    f32 = jnp.float32
    pad = EP - E
    srcb = jnp.concatenate(
        [edge_index[0], jnp.zeros((pad,), jnp.int32)]).reshape(NW, TPC, CH)
    dstb = jnp.concatenate(
        [edge_index[1], jnp.full((pad,), N, jnp.int32)]).reshape(NW, TPC, CH)
    xp = jnp.pad(x, ((0, NP - N), (0, 0)))
    zeros64 = jnp.zeros((NP, 64), f32)
    zeros16 = jnp.zeros((NP, 16), f32)
    ones16 = jnp.ones((CH, 16), f32)

    degp = _deg_sc(dstb, zeros16, ones16)
    hs1, dinv16 = pl.pallas_call(
        _tc_first, out_shape=(_f32(NP, 64), _f32(NP, 16)))(xp, W1, degp)
    p1 = _agg_sc(hs1, srcb, dstb, zeros64)
    hs2 = pl.pallas_call(_tc_mid, out_shape=_f32(NP, 64))(
        p1, hs1, dinv16, b1.reshape(1, 64), W2)
    p2 = _agg_sc(hs2, srcb, dstb, zeros64)
    hs3 = pl.pallas_call(_tc_mid, out_shape=_f32(NP, 64))(
        p2, hs2, dinv16, b2.reshape(1, 64), W3)
    p3 = _agg_sc(hs3, srcb, dstb, zeros64)
    hs4 = pl.pallas_call(_tc_pre4, out_shape=_f32(NP, 64))(
        p3, hs3, dinv16, b3.reshape(1, 64))
    p4 = _agg_sc(hs4, srcb, dstb, zeros64)
    outp = pl.pallas_call(_tc_last, out_shape=_f32(NP, 128))(
        p4, hs4, dinv16, W4, b4.reshape(1, 128))
    return outp[:N]


# speedup vs baseline: 28.5683x; 2.2198x over previous
import functools

import jax
import jax.numpy as jnp
from jax import lax
from jax.experimental import pallas as pl
from jax.experimental.pallas import tpu as pltpu
from jax.experimental.pallas import tpu_sc as plsc

N = 10000
NP = 10112
E = 320000
NW = 32
CH = 128
TPC = 80
EP = NW * TPC * CH
RS = NP // 16

_mesh = plsc.VectorSubcoreMesh(core_axis_name="c", subcore_axis_name="s")
_sc_params = pltpu.CompilerParams(use_tc_tiling_on_sc=False)


@functools.partial(
    pl.kernel,
    out_type=jax.ShapeDtypeStruct((2, NP, 16), jnp.float32),
    mesh=_mesh,
    compiler_params=_sc_params,
    scratch_types=[
        pltpu.VMEM((TPC, CH), jnp.int32),
        pltpu.VMEM((CH, 16), jnp.float32),
        pltpu.VMEM_SHARED((NP, 16), jnp.float32),
    ],
)
def _deg_sc(dst_hbm, zeros_hbm, ones_hbm, out_hbm, dstv, onesv, deg_sp):
    c = lax.axis_index("c")
    s = lax.axis_index("s")
    w = c * 16 + s
    pltpu.sync_copy(zeros_hbm.at[pl.ds(s * RS, RS)], deg_sp.at[pl.ds(s * RS, RS)])
    pltpu.sync_copy(ones_hbm, onesv)
    pltpu.sync_copy(dst_hbm.at[w], dstv)
    plsc.subcore_barrier()

    def body(j, carry):
        pltpu.sync_copy(onesv, deg_sp.at[dstv.at[j]], add=True)
        return carry

    lax.fori_loop(0, TPC, body, 0)
    plsc.subcore_barrier()
    pltpu.sync_copy(deg_sp.at[pl.ds(s * RS, RS)], out_hbm.at[c, pl.ds(s * RS, RS)])


@functools.partial(
    pl.kernel,
    out_type=jax.ShapeDtypeStruct((2, NP, 64), jnp.float32),
    mesh=_mesh,
    compiler_params=_sc_params,
    scratch_types=[
        pltpu.VMEM((TPC, CH), jnp.int32),
        pltpu.VMEM((TPC, CH), jnp.int32),
        pltpu.VMEM((2, CH, 64), jnp.float32),
        pltpu.VMEM_SHARED((NP, 64), jnp.float32),
        pltpu.VMEM_SHARED((NP, 64), jnp.float32),
        pltpu.SemaphoreType.DMA,
        pltpu.SemaphoreType.DMA,
    ],
)
def _agg_sc(hs_hbm, src_hbm, dst_hbm, zeros_hbm, out_hbm,
            srcv, dstv, rows, agg_sp, hs_sp, sem0, sem1):
    c = lax.axis_index("c")
    s = lax.axis_index("s")
    w = c * 16 + s
    pltpu.sync_copy(zeros_hbm.at[pl.ds(s * RS, RS)], agg_sp.at[pl.ds(s * RS, RS)])
    pltpu.sync_copy(hs_hbm.at[pl.ds(s * RS, RS)], hs_sp.at[pl.ds(s * RS, RS)])
    pltpu.sync_copy(src_hbm.at[w], srcv)
    pltpu.sync_copy(dst_hbm.at[w], dstv)
    plsc.subcore_barrier()

    sems = (sem0, sem1)

    def gather(j, b):
        return pltpu.make_async_copy(hs_sp.at[srcv.at[j]], rows.at[b], sems[b])

    gather(0, 0).start()
    gather(1, 1).start()

    def body(i, carry):
        for b in range(2):
            j = i * 2 + b
            gather(j, b).wait()
            pltpu.sync_copy(rows.at[b], agg_sp.at[dstv.at[j]], add=True)

            @pl.when(j + 2 < TPC)
            def _():
                gather(j + 2, b).start()
        return carry

    lax.fori_loop(0, TPC // 2, body, 0)
    plsc.subcore_barrier()
    pltpu.sync_copy(agg_sp.at[pl.ds(s * RS, RS)], out_hbm.at[c, pl.ds(s * RS, RS)])


def _tc_first(x_ref, w_ref, degp_ref, hs_ref, dinv_ref):
    deg = degp_ref[0, :, 0:1] + degp_ref[1, :, 0:1] + 1.0
    rows = lax.broadcasted_iota(jnp.int32, (NP, 1), 0)
    dinv = jnp.where(rows < N, lax.rsqrt(deg), 0.0)
    t = jnp.dot(x_ref[...], w_ref[...], preferred_element_type=jnp.float32)
    hs_ref[...] = t * dinv
    dinv_ref[...] = jnp.broadcast_to(dinv, (NP, 16))


def _tc_mid(p_ref, hs_ref, dinv_ref, b_ref, w_ref, out_ref):
    dinv = dinv_ref[:, 0:1]
    agg = p_ref[0] + p_ref[1] + hs_ref[...]
    h = jnp.maximum(agg * dinv + b_ref[...], 0.0)
    out_ref[...] = jnp.dot(h, w_ref[...], preferred_element_type=jnp.float32) * dinv


def _tc_pre4(p_ref, hs_ref, dinv_ref, b_ref, out_ref):
    dinv = dinv_ref[:, 0:1]
    agg = p_ref[0] + p_ref[1] + hs_ref[...]
    out_ref[...] = jnp.maximum(agg * dinv + b_ref[...], 0.0) * dinv


def _tc_last(p_ref, hs_ref, dinv_ref, w_ref, b_ref, out_ref):
    dinv = dinv_ref[:, 0:1]
    agg = (p_ref[0] + p_ref[1] + hs_ref[...]) * dinv
    out_ref[...] = jnp.dot(agg, w_ref[...], preferred_element_type=jnp.float32) + b_ref[...]


def _f32(*shape):
    return jax.ShapeDtypeStruct(shape, jnp.float32)


def kernel(x, edge_index, W1, b1, W2, b2, W3, b3, W4, b4):
    f32 = jnp.float32
    pad = EP - E
    srcb = jnp.concatenate(
        [edge_index[0], jnp.zeros((pad,), jnp.int32)]).reshape(NW, TPC, CH)
    dstb = jnp.concatenate(
        [edge_index[1], jnp.full((pad,), N, jnp.int32)]).reshape(NW, TPC, CH)
    xp = jnp.pad(x, ((0, NP - N), (0, 0)))
    zeros64 = jnp.zeros((NP, 64), f32)
    zeros16 = jnp.zeros((NP, 16), f32)
    ones16 = jnp.ones((CH, 16), f32)

    degp = _deg_sc(dstb, zeros16, ones16)
    hs1, dinv16 = pl.pallas_call(
        _tc_first, out_shape=(_f32(NP, 64), _f32(NP, 16)))(xp, W1, degp)
    p1 = _agg_sc(hs1, srcb, dstb, zeros64)
    hs2 = pl.pallas_call(_tc_mid, out_shape=_f32(NP, 64))(
        p1, hs1, dinv16, b1.reshape(1, 64), W2)
    p2 = _agg_sc(hs2, srcb, dstb, zeros64)
    hs3 = pl.pallas_call(_tc_mid, out_shape=_f32(NP, 64))(
        p2, hs2, dinv16, b2.reshape(1, 64), W3)
    p3 = _agg_sc(hs3, srcb, dstb, zeros64)
    hs4 = pl.pallas_call(_tc_pre4, out_shape=_f32(NP, 64))(
        p3, hs3, dinv16, b3.reshape(1, 64))
    p4 = _agg_sc(hs4, srcb, dstb, zeros64)
    outp = pl.pallas_call(_tc_last, out_shape=_f32(NP, 128))(
        p4, hs4, dinv16, W4, b4.reshape(1, 128))
    return outp[:N]
